# baseline (speedup 1.0000x reference)
import functools

import jax
import jax.numpy as jnp
from jax import lax
from jax.experimental import pallas as pl
from jax.experimental.pallas import tpu as pltpu
from jax.experimental.pallas import tpu_sc as plsc

N_NODES = 10000
N_EDGES = 320000
D = 128
N_PAD = 10240
E_ROWS = N_EDGES // 128
E_ROWS_PAD = 2560
NC = 2
NS = 16
NW = NC * NS
TILE_ROWS = E_ROWS_PAD // NW
ROWS_PER_TILE = N_PAD // NS

_sc_mesh = plsc.VectorSubcoreMesh(core_axis_name="c", subcore_axis_name="s")
_sc_params = pltpu.CompilerParams(needs_layout_passes=False)


@functools.partial(
    pl.kernel,
    out_type=jax.ShapeDtypeStruct((2 * NW, N_PAD), jnp.float32),
    mesh=_sc_mesh,
    compiler_params=_sc_params,
    scratch_types=[
        pltpu.VMEM((TILE_ROWS, 128), jnp.int32),
        pltpu.VMEM((TILE_ROWS, 128), jnp.int32),
        pltpu.VMEM((N_PAD,), jnp.float32),
        pltpu.VMEM((N_PAD,), jnp.float32),
    ],
)
def _degree_kernel(src_hbm, dst_hbm, out_hbm, src_v, dst_v, dego_v, degi_v):
    cid = lax.axis_index("c")
    sid = lax.axis_index("s")
    wid = sid * NC + cid
    base = wid * TILE_ROWS

    pltpu.sync_copy(src_hbm.at[pl.ds(base, TILE_ROWS)], src_v)
    pltpu.sync_copy(dst_hbm.at[pl.ds(base, TILE_ROWS)], dst_v)

    zeros16 = jnp.zeros((16,), jnp.float32)

    def zero_body(k, _):
        dego_v[pl.ds(k * 16, 16)] = zeros16
        degi_v[pl.ds(k * 16, 16)] = zeros16
        return 0

    lax.fori_loop(0, N_PAD // 16, zero_body, 0, unroll=8)

    ones16 = jnp.ones((16,), jnp.float32)

    def row_body(i, _):
        for j in range(8):
            s16 = src_v[i, pl.ds(j * 16, 16)]
            plsc.addupdate_scatter(dego_v, [s16], ones16)
            d16 = dst_v[i, pl.ds(j * 16, 16)]
            plsc.addupdate_scatter(degi_v, [d16], ones16)
        return 0

    lax.fori_loop(0, TILE_ROWS, row_body, 0, unroll=4)

    pltpu.sync_copy(dego_v, out_hbm.at[wid])
    pltpu.sync_copy(degi_v, out_hbm.at[NW + wid])


def _norm_body(deg_ref, out_ref):
    d = deg_ref[...]
    deg_out = jnp.sum(d[:NW, :], axis=0, keepdims=True)
    deg_in = jnp.sum(d[NW:, :], axis=0, keepdims=True)
    s = jnp.concatenate([deg_out, deg_in], axis=0)
    out_ref[...] = lax.rsqrt(jnp.maximum(s, 1.0))


def _norms(deg_partial):
    return pl.pallas_call(
        _norm_body,
        out_shape=jax.ShapeDtypeStruct((2, N_PAD), jnp.float32),
    )(deg_partial)


@functools.partial(
    pl.kernel,
    out_type=(
        jax.ShapeDtypeStruct((N_PAD, D), jnp.float32),
        jax.ShapeDtypeStruct((N_PAD, D), jnp.float32),
    ),
    mesh=_sc_mesh,
    compiler_params=_sc_params,
    scratch_types=[
        pltpu.VMEM((N_PAD,), jnp.float32),
        pltpu.VMEM((2, 128), jnp.int32),
        pltpu.VMEM((2, 128), jnp.int32),
        pltpu.VMEM((2, 128), jnp.float32),
        pltpu.VMEM((128,), jnp.float32),
        pltpu.VMEM((2, 128, D), jnp.float32),
        pltpu.VMEM_SHARED((N_PAD, D), jnp.float32),
        pltpu.SemaphoreType.DMA,
        pltpu.SemaphoreType.DMA,
        pltpu.SemaphoreType.DMA,
        pltpu.SemaphoreType.DMA,
    ],
)
def _agg_kernel(x_hbm, src_hbm, dst_hbm, w_hbm, norm_hbm,
                out0_hbm, out1_hbm, norm_v, src_v, dst_v, w_v, wp_v,
                rows_v, agg_sh, semi0, semi1, semg0, semg1):
    cid = lax.axis_index("c")
    sid = lax.axis_index("s")
    wid = sid * NC + cid
    base = wid * TILE_ROWS

    pltpu.sync_copy(norm_hbm, norm_v)

    zeros16 = jnp.zeros((16,), jnp.float32)

    def zero_body(k, _):
        rows_v[0, k // 8, pl.ds((k % 8) * 16, 16)] = zeros16
        return 0

    lax.fori_loop(0, 128 * 8, zero_body, 0, unroll=8)
    for b in range(ROWS_PER_TILE // 128):
        pltpu.sync_copy(rows_v.at[0],
                        agg_sh.at[pl.ds(sid * ROWS_PER_TILE + b * 128, 128)])
    plsc.subcore_barrier()

    semi = (semi0, semi1)
    semg = (semg0, semg1)

    def issue_idx(slot, r, sem):
        pltpu.async_copy(src_hbm.at[r], src_v.at[slot], sem)
        pltpu.async_copy(dst_hbm.at[r], dst_v.at[slot], sem)
        pltpu.async_copy(w_hbm.at[r], w_v.at[slot], sem)

    def wait_idx(slot, sem):
        pltpu.make_async_copy(src_hbm.at[0], src_v.at[slot], sem).wait()
        pltpu.make_async_copy(dst_hbm.at[0], dst_v.at[slot], sem).wait()
        pltpu.make_async_copy(w_hbm.at[0], w_v.at[slot], sem).wait()

    def scale_weights(slot):
        for j in range(8):
            idx16 = src_v[slot, pl.ds(j * 16, 16)]
            nrm16 = plsc.load_gather(norm_v, [idx16])
            wp_v[pl.ds(j * 16, 16)] = w_v[slot, pl.ds(j * 16, 16)] * nrm16

    def scale_and_scatter(slot):
        def edge_body(e, _):
            wsc = plsc.load_gather(wp_v, [jnp.full((16,), e, jnp.int32)])
            for f in range(8):
                rows_v[slot, e, pl.ds(f * 16, 16)] = (
                    rows_v[slot, e, pl.ds(f * 16, 16)] * wsc)
            return 0

        lax.fori_loop(0, 128, edge_body, 0, unroll=8)
        pltpu.sync_copy(rows_v.at[slot], agg_sh.at[dst_v.at[slot]], add=True)

    n_pairs = TILE_ROWS // 2
    issue_idx(0, base, semi0)
    wait_idx(0, semi0)
    pltpu.async_copy(x_hbm.at[src_v.at[0]], rows_v.at[0], semg0)

    def pair_body(g, _):
        r0 = base + 2 * g
        issue_idx(1, r0 + 1, semi1)
        pltpu.make_async_copy(x_hbm.at[src_v.at[0]], rows_v.at[0],
                              semg0).wait()
        scale_weights(0)
        wait_idx(1, semi1)
        pltpu.async_copy(x_hbm.at[src_v.at[1]], rows_v.at[1], semg1)
        scale_and_scatter(0)

        @pl.when(g + 1 < n_pairs)
        def _():
            issue_idx(0, r0 + 2, semi0)

        pltpu.make_async_copy(x_hbm.at[src_v.at[1]], rows_v.at[1],
                              semg1).wait()
        scale_weights(1)

        @pl.when(g + 1 < n_pairs)
        def _():
            wait_idx(0, semi0)
            pltpu.async_copy(x_hbm.at[src_v.at[0]], rows_v.at[0], semg0)

        scale_and_scatter(1)
        return 0

    lax.fori_loop(0, n_pairs, pair_body, 0)
    plsc.subcore_barrier()

    @pl.when(cid == 0)
    def _():
        for b in range(ROWS_PER_TILE // 128):
            off = sid * ROWS_PER_TILE + b * 128
            pltpu.sync_copy(agg_sh.at[pl.ds(off, 128)], out0_hbm.at[pl.ds(off, 128)])

    @pl.when(cid == 1)
    def _():
        for b in range(ROWS_PER_TILE // 128):
            off = sid * ROWS_PER_TILE + b * 128
            pltpu.sync_copy(agg_sh.at[pl.ds(off, 128)], out1_hbm.at[pl.ds(off, 128)])


def _mlp_body(a0_ref, a1_ref, nd_ref, wc_ref, bc_ref, wf_ref, bf_ref,
              w2_ref, b2_ref, out_ref):
    h = (a0_ref[...] + a1_ref[...]) * nd_ref[...]
    h = jnp.dot(h, wc_ref[...], preferred_element_type=jnp.float32) + bc_ref[...]
    h = jnp.maximum(h, 0.0)
    h = jnp.dot(h, wf_ref[...], preferred_element_type=jnp.float32) + bf_ref[...]
    h = jnp.maximum(h, 0.0)
    out_ref[...] = (
        jnp.dot(h, w2_ref[...], preferred_element_type=jnp.float32) + b2_ref[...]
    )


def _mlp(a0, a1, norm_dst, W_conv, b_conv, W_fc, b_fc, W_fc2, b_fc2):
    BR = 1000
    grid = (N_NODES // BR,)
    row_spec = pl.BlockSpec((BR, D), lambda i: (i, 0))
    nd_spec = pl.BlockSpec((BR, 1), lambda i: (i, 0))
    w_spec = pl.BlockSpec((D, D), lambda i: (0, 0))
    b_spec = pl.BlockSpec((1, D), lambda i: (0, 0))
    return pl.pallas_call(
        _mlp_body,
        grid=grid,
        in_specs=[row_spec, row_spec, nd_spec, w_spec, b_spec, w_spec,
                  b_spec, w_spec, b_spec],
        out_specs=row_spec,
        out_shape=jax.ShapeDtypeStruct((N_NODES, D), jnp.float32),
    )(a0, a1, norm_dst, W_conv, b_conv, W_fc, b_fc, W_fc2, b_fc2)


@jax.jit
def kernel(x, edge_index, edge_weight, W_conv, b_conv, W_fc, b_fc, W_fc2,
           b_fc2):
    n_pad_edges = E_ROWS_PAD * 128 - N_EDGES
    src = edge_index[0].astype(jnp.int32)
    dst = edge_index[1].astype(jnp.int32)
    pad_node = jnp.full((n_pad_edges,), N_PAD - 1, jnp.int32)
    src_deg2d = jnp.concatenate([src, pad_node]).reshape(E_ROWS_PAD, 128)
    dst2d = jnp.concatenate([dst, pad_node]).reshape(E_ROWS_PAD, 128)
    src_agg2d = jnp.concatenate(
        [src, jnp.zeros((n_pad_edges,), jnp.int32)]).reshape(E_ROWS_PAD, 128)
    w2d = jnp.concatenate(
        [edge_weight, jnp.zeros((n_pad_edges,), jnp.float32)]
    ).reshape(E_ROWS_PAD, 128)

    deg_partial = _degree_kernel(src_deg2d, dst2d)
    norms = _norms(deg_partial)
    agg0, agg1 = _agg_kernel(x, src_agg2d, dst2d, w2d, norms[0])
    return _mlp(agg0[:N_NODES], agg1[:N_NODES],
                norms[1, :N_NODES, None], W_conv,
                b_conv.reshape(1, D), W_fc, b_fc.reshape(1, D), W_fc2,
                b_fc2.reshape(1, D))


# speedup vs baseline: 1.0066x; 1.0066x over previous
import functools

import jax
import jax.numpy as jnp
from jax import lax
from jax.experimental import pallas as pl
from jax.experimental.pallas import tpu as pltpu
from jax.experimental.pallas import tpu_sc as plsc

N_NODES = 10000
N_EDGES = 320000
D = 128
N_PAD = 10240
E_ROWS = N_EDGES // 128
E_ROWS_PAD = 2560
NC = 2
NS = 16
NW = NC * NS
TILE_ROWS = E_ROWS_PAD // NW
ROWS_PER_TILE = N_PAD // NS

_sc_mesh = plsc.VectorSubcoreMesh(core_axis_name="c", subcore_axis_name="s")
_sc_params = pltpu.CompilerParams(needs_layout_passes=False)


@functools.partial(
    pl.kernel,
    out_type=jax.ShapeDtypeStruct((2 * NW, N_PAD), jnp.float32),
    mesh=_sc_mesh,
    compiler_params=_sc_params,
    scratch_types=[
        pltpu.VMEM((TILE_ROWS, 128), jnp.int32),
        pltpu.VMEM((TILE_ROWS, 128), jnp.int32),
        pltpu.VMEM((N_PAD,), jnp.float32),
        pltpu.VMEM((N_PAD,), jnp.float32),
    ],
)
def _degree_kernel(src_hbm, dst_hbm, out_hbm, src_v, dst_v, dego_v, degi_v):
    cid = lax.axis_index("c")
    sid = lax.axis_index("s")
    wid = sid * NC + cid
    base = wid * TILE_ROWS

    pltpu.sync_copy(src_hbm.at[pl.ds(base, TILE_ROWS)], src_v)
    pltpu.sync_copy(dst_hbm.at[pl.ds(base, TILE_ROWS)], dst_v)

    zeros16 = jnp.zeros((16,), jnp.float32)

    def zero_body(k, _):
        dego_v[pl.ds(k * 16, 16)] = zeros16
        degi_v[pl.ds(k * 16, 16)] = zeros16
        return 0

    lax.fori_loop(0, N_PAD // 16, zero_body, 0, unroll=8)

    ones16 = jnp.ones((16,), jnp.float32)

    def row_body(i, _):
        for j in range(8):
            s16 = src_v[i, pl.ds(j * 16, 16)]
            plsc.addupdate_scatter(dego_v, [s16], ones16)
            d16 = dst_v[i, pl.ds(j * 16, 16)]
            plsc.addupdate_scatter(degi_v, [d16], ones16)
        return 0

    lax.fori_loop(0, TILE_ROWS, row_body, 0, unroll=4)

    pltpu.sync_copy(dego_v, out_hbm.at[wid])
    pltpu.sync_copy(degi_v, out_hbm.at[NW + wid])


def _norm_body(deg_ref, out_ref):
    d = deg_ref[...]
    deg_out = jnp.sum(d[:NW, :], axis=0, keepdims=True)
    deg_in = jnp.sum(d[NW:, :], axis=0, keepdims=True)
    s = jnp.concatenate([deg_out, deg_in], axis=0)
    out_ref[...] = lax.rsqrt(jnp.maximum(s, 1.0))


def _norms(deg_partial):
    return pl.pallas_call(
        _norm_body,
        out_shape=jax.ShapeDtypeStruct((2, N_PAD), jnp.float32),
    )(deg_partial)


@functools.partial(
    pl.kernel,
    out_type=(
        jax.ShapeDtypeStruct((N_PAD, D), jnp.float32),
        jax.ShapeDtypeStruct((N_PAD, D), jnp.float32),
    ),
    mesh=_sc_mesh,
    compiler_params=_sc_params,
    scratch_types=[
        pltpu.VMEM((N_PAD,), jnp.float32),
        pltpu.VMEM((2, 128), jnp.int32),
        pltpu.VMEM((4, 128), jnp.int32),
        pltpu.VMEM((2, 128), jnp.float32),
        pltpu.VMEM((128,), jnp.float32),
        pltpu.VMEM((2, 128, D), jnp.float32),
        pltpu.VMEM_SHARED((N_PAD, D), jnp.float32),
        pltpu.SemaphoreType.DMA,
        pltpu.SemaphoreType.DMA,
        pltpu.SemaphoreType.DMA,
        pltpu.SemaphoreType.DMA,
        pltpu.SemaphoreType.DMA,
        pltpu.SemaphoreType.DMA,
    ],
)
def _agg_kernel(x_hbm, src_hbm, dst_hbm, w_hbm, norm_hbm,
                out0_hbm, out1_hbm, norm_v, src_v, dst_v, w_v, wp_v,
                rows_v, agg_sh, semi0, semi1, semg0, semg1, sems0, sems1):
    cid = lax.axis_index("c")
    sid = lax.axis_index("s")
    wid = sid * NC + cid
    base = wid * TILE_ROWS

    pltpu.sync_copy(norm_hbm, norm_v)

    zeros16 = jnp.zeros((16,), jnp.float32)

    def zero_body(k, _):
        rows_v[0, k // 8, pl.ds((k % 8) * 16, 16)] = zeros16
        return 0

    lax.fori_loop(0, 128 * 8, zero_body, 0, unroll=8)
    for b in range(ROWS_PER_TILE // 128):
        pltpu.sync_copy(rows_v.at[0],
                        agg_sh.at[pl.ds(sid * ROWS_PER_TILE + b * 128, 128)])
    plsc.subcore_barrier()

    semi = (semi0, semi1)
    semg = (semg0, semg1)
    sems = (sems0, sems1)
    n_quads = TILE_ROWS // 4

    def issue_idx(s, dk, r, sem):
        pltpu.async_copy(src_hbm.at[r], src_v.at[s], sem)
        pltpu.async_copy(dst_hbm.at[r], dst_v.at[dk], sem)
        pltpu.async_copy(w_hbm.at[r], w_v.at[s], sem)

    def wait_idx(s, dk, sem):
        pltpu.make_async_copy(src_hbm.at[0], src_v.at[s], sem).wait()
        pltpu.make_async_copy(dst_hbm.at[0], dst_v.at[dk], sem).wait()
        pltpu.make_async_copy(w_hbm.at[0], w_v.at[s], sem).wait()

    def wait_scatter(par):
        pltpu.make_async_copy(rows_v.at[0], agg_sh.at[dst_v.at[0]],
                              sems[par]).wait()

    def scale_weights(s):
        for j in range(8):
            idx16 = src_v[s, pl.ds(j * 16, 16)]
            nrm16 = plsc.load_gather(norm_v, [idx16])
            wp_v[pl.ds(j * 16, 16)] = w_v[s, pl.ds(j * 16, 16)] * nrm16

    def edge_scale(s):
        def edge_body(e, _):
            wsc = plsc.load_gather(wp_v, [jnp.full((16,), e, jnp.int32)])
            for f in range(8):
                rows_v[s, e, pl.ds(f * 16, 16)] = (
                    rows_v[s, e, pl.ds(f * 16, 16)] * wsc)
            return 0

        lax.fori_loop(0, 128, edge_body, 0, unroll=8)

    issue_idx(0, 0, base, semi0)
    wait_idx(0, 0, semi0)
    pltpu.async_copy(x_hbm.at[src_v.at[0]], rows_v.at[0], semg0)
    issue_idx(1, 1, base + 1, semi1)

    def quad_body(q, _):
        c0 = base + 4 * q
        for k in range(4):
            s, sn, dk = k % 2, (k + 1) % 2, k
            pltpu.make_async_copy(x_hbm.at[src_v.at[s]], rows_v.at[s],
                                  semg[s]).wait()
            scale_weights(s)
            if k < 2:
                issue_idx(s, (k + 2) % 4, c0 + k + 2, semi[s])
            else:
                @pl.when(q + 1 < n_quads)
                def _():
                    issue_idx(s, (k + 2) % 4, c0 + k + 2, semi[s])
            if k == 3:
                @pl.when(q + 1 < n_quads)
                def _():
                    wait_idx(sn, (k + 1) % 4, semi[sn])
                    pltpu.async_copy(x_hbm.at[src_v.at[sn]], rows_v.at[sn],
                                     semg[sn])
            else:
                wait_idx(sn, (k + 1) % 4, semi[sn])
                pltpu.async_copy(x_hbm.at[src_v.at[sn]], rows_v.at[sn],
                                 semg[sn])
            edge_scale(s)
            pltpu.async_copy(rows_v.at[s], agg_sh.at[dst_v.at[dk]], sems[s],
                             add=True).wait()
        return 0

    lax.fori_loop(0, n_quads, quad_body, 0)
    plsc.subcore_barrier()

    @pl.when(cid == 0)
    def _():
        for b in range(ROWS_PER_TILE // 128):
            off = sid * ROWS_PER_TILE + b * 128
            pltpu.sync_copy(agg_sh.at[pl.ds(off, 128)], out0_hbm.at[pl.ds(off, 128)])

    @pl.when(cid == 1)
    def _():
        for b in range(ROWS_PER_TILE // 128):
            off = sid * ROWS_PER_TILE + b * 128
            pltpu.sync_copy(agg_sh.at[pl.ds(off, 128)], out1_hbm.at[pl.ds(off, 128)])


def _mlp_body(a0_ref, a1_ref, nd_ref, wc_ref, bc_ref, wf_ref, bf_ref,
              w2_ref, b2_ref, out_ref):
    h = (a0_ref[...] + a1_ref[...]) * nd_ref[...]
    h = jnp.dot(h, wc_ref[...], preferred_element_type=jnp.float32) + bc_ref[...]
    h = jnp.maximum(h, 0.0)
    h = jnp.dot(h, wf_ref[...], preferred_element_type=jnp.float32) + bf_ref[...]
    h = jnp.maximum(h, 0.0)
    out_ref[...] = (
        jnp.dot(h, w2_ref[...], preferred_element_type=jnp.float32) + b2_ref[...]
    )


def _mlp(a0, a1, norm_dst, W_conv, b_conv, W_fc, b_fc, W_fc2, b_fc2):
    BR = 1000
    grid = (N_NODES // BR,)
    row_spec = pl.BlockSpec((BR, D), lambda i: (i, 0))
    nd_spec = pl.BlockSpec((BR, 1), lambda i: (i, 0))
    w_spec = pl.BlockSpec((D, D), lambda i: (0, 0))
    b_spec = pl.BlockSpec((1, D), lambda i: (0, 0))
    return pl.pallas_call(
        _mlp_body,
        grid=grid,
        in_specs=[row_spec, row_spec, nd_spec, w_spec, b_spec, w_spec,
                  b_spec, w_spec, b_spec],
        out_specs=row_spec,
        out_shape=jax.ShapeDtypeStruct((N_NODES, D), jnp.float32),
    )(a0, a1, norm_dst, W_conv, b_conv, W_fc, b_fc, W_fc2, b_fc2)


@jax.jit
def kernel(x, edge_index, edge_weight, W_conv, b_conv, W_fc, b_fc, W_fc2,
           b_fc2):
    n_pad_edges = E_ROWS_PAD * 128 - N_EDGES
    src = edge_index[0].astype(jnp.int32)
    dst = edge_index[1].astype(jnp.int32)
    pad_node = jnp.full((n_pad_edges,), N_PAD - 1, jnp.int32)
    src_deg2d = jnp.concatenate([src, pad_node]).reshape(E_ROWS_PAD, 128)
    dst2d = jnp.concatenate([dst, pad_node]).reshape(E_ROWS_PAD, 128)
    src_agg2d = jnp.concatenate(
        [src, jnp.zeros((n_pad_edges,), jnp.int32)]).reshape(E_ROWS_PAD, 128)
    w2d = jnp.concatenate(
        [edge_weight, jnp.zeros((n_pad_edges,), jnp.float32)]
    ).reshape(E_ROWS_PAD, 128)

    deg_partial = _degree_kernel(src_deg2d, dst2d)
    norms = _norms(deg_partial)
    agg0, agg1 = _agg_kernel(x, src_agg2d, dst2d, w2d, norms[0])
    return _mlp(agg0[:N_NODES], agg1[:N_NODES],
                norms[1, :N_NODES, None], W_conv,
                b_conv.reshape(1, D), W_fc, b_fc.reshape(1, D), W_fc2,
                b_fc2.reshape(1, D))


# speedup vs baseline: 1.3674x; 1.3585x over previous
import functools

import jax
import jax.numpy as jnp
from jax import lax
from jax.experimental import pallas as pl
from jax.experimental.pallas import tpu as pltpu
from jax.experimental.pallas import tpu_sc as plsc

N_NODES = 10000
N_EDGES = 320000
D = 128
N_PAD = 10240
E_ROWS = N_EDGES // 128
E_ROWS_PAD = 2560
NC = 2
NS = 16
NW = NC * NS
TILE_ROWS = E_ROWS_PAD // NW
ROWS_PER_TILE = N_PAD // NS

_sc_mesh = plsc.VectorSubcoreMesh(core_axis_name="c", subcore_axis_name="s")
_sc_params = pltpu.CompilerParams(needs_layout_passes=False)
_sc_params_untiled = pltpu.CompilerParams(needs_layout_passes=False,
                                          use_tc_tiling_on_sc=False)

import numpy as _np
_BF16_PERM = _np.concatenate([
    32 * f + _np.stack([_np.arange(16), 16 + _np.arange(16)], axis=1).ravel()
    for f in range(4)
])


@functools.partial(
    pl.kernel,
    out_type=jax.ShapeDtypeStruct((2 * NW, N_PAD), jnp.float32),
    mesh=_sc_mesh,
    compiler_params=_sc_params,
    scratch_types=[
        pltpu.VMEM((TILE_ROWS, 128), jnp.int32),
        pltpu.VMEM((TILE_ROWS, 128), jnp.int32),
        pltpu.VMEM((N_PAD,), jnp.float32),
        pltpu.VMEM((N_PAD,), jnp.float32),
    ],
)
def _degree_kernel(src_hbm, dst_hbm, out_hbm, src_v, dst_v, dego_v, degi_v):
    cid = lax.axis_index("c")
    sid = lax.axis_index("s")
    wid = sid * NC + cid
    base = wid * TILE_ROWS

    pltpu.sync_copy(src_hbm.at[pl.ds(base, TILE_ROWS)], src_v)
    pltpu.sync_copy(dst_hbm.at[pl.ds(base, TILE_ROWS)], dst_v)

    zeros16 = jnp.zeros((16,), jnp.float32)

    def zero_body(k, _):
        dego_v[pl.ds(k * 16, 16)] = zeros16
        degi_v[pl.ds(k * 16, 16)] = zeros16
        return 0

    lax.fori_loop(0, N_PAD // 16, zero_body, 0, unroll=8)

    ones16 = jnp.ones((16,), jnp.float32)

    def row_body(i, _):
        for j in range(8):
            s16 = src_v[i, pl.ds(j * 16, 16)]
            plsc.addupdate_scatter(dego_v, [s16], ones16)
            d16 = dst_v[i, pl.ds(j * 16, 16)]
            plsc.addupdate_scatter(degi_v, [d16], ones16)
        return 0

    lax.fori_loop(0, TILE_ROWS, row_body, 0, unroll=4)

    pltpu.sync_copy(dego_v, out_hbm.at[wid])
    pltpu.sync_copy(degi_v, out_hbm.at[NW + wid])


def _norm_body(deg_ref, out_ref):
    d = deg_ref[...]
    deg_out = jnp.sum(d[:NW, :], axis=0, keepdims=True)
    deg_in = jnp.sum(d[NW:, :], axis=0, keepdims=True)
    s = jnp.concatenate([deg_out, deg_in], axis=0)
    out_ref[...] = lax.rsqrt(jnp.maximum(s, 1.0))


def _norms(deg_partial):
    return pl.pallas_call(
        _norm_body,
        out_shape=jax.ShapeDtypeStruct((2, N_PAD), jnp.float32),
    )(deg_partial)


@functools.partial(
    pl.kernel,
    out_type=(
        jax.ShapeDtypeStruct((N_PAD, D), jnp.float32),
        jax.ShapeDtypeStruct((N_PAD, D), jnp.float32),
    ),
    mesh=_sc_mesh,
    compiler_params=_sc_params_untiled,
    scratch_types=[
        pltpu.VMEM((N_PAD,), jnp.float32),
        pltpu.VMEM((2, 128), jnp.int32),
        pltpu.VMEM((4, 128), jnp.int32),
        pltpu.VMEM((2, 128), jnp.float32),
        pltpu.VMEM((128,), jnp.float32),
        pltpu.VMEM((2, 128, D // 2), jnp.int32),
        pltpu.VMEM((128, D), jnp.float32),
        pltpu.VMEM_SHARED((N_PAD, D), jnp.float32),
        pltpu.SemaphoreType.DMA,
        pltpu.SemaphoreType.DMA,
        pltpu.SemaphoreType.DMA,
        pltpu.SemaphoreType.DMA,
        pltpu.SemaphoreType.DMA,
        pltpu.SemaphoreType.DMA,
    ],
)
def _agg_kernel(x_hbm, src_hbm, dst_hbm, w_hbm, norm_hbm,
                out0_hbm, out1_hbm, norm_v, src_v, dst_v, w_v, wp_v,
                rows_v, rowf_v, agg_sh, semi0, semi1, semg0, semg1,
                sems0, sems1):
    cid = lax.axis_index("c")
    sid = lax.axis_index("s")
    wid = sid * NC + cid
    base = wid * TILE_ROWS

    pltpu.sync_copy(norm_hbm, norm_v)

    zeros16 = jnp.zeros((16,), jnp.float32)

    def zero_body(k, _):
        rowf_v[k // 8, pl.ds((k % 8) * 16, 16)] = zeros16
        return 0

    lax.fori_loop(0, 128 * 8, zero_body, 0, unroll=8)
    for b in range(ROWS_PER_TILE // 128):
        pltpu.sync_copy(rowf_v,
                        agg_sh.at[pl.ds(sid * ROWS_PER_TILE + b * 128, 128)])
    plsc.subcore_barrier()

    semi = (semi0, semi1)
    semg = (semg0, semg1)
    sems = (sems0, sems1)
    n_quads = TILE_ROWS // 4

    def issue_idx(s, dk, r, sem):
        pltpu.async_copy(src_hbm.at[r], src_v.at[s], sem)
        pltpu.async_copy(dst_hbm.at[r], dst_v.at[dk], sem)
        pltpu.async_copy(w_hbm.at[r], w_v.at[s], sem)

    def wait_idx(s, dk, sem):
        pltpu.make_async_copy(src_hbm.at[0], src_v.at[s], sem).wait()
        pltpu.make_async_copy(dst_hbm.at[0], dst_v.at[dk], sem).wait()
        pltpu.make_async_copy(w_hbm.at[0], w_v.at[s], sem).wait()

    def wait_scatter(par):
        pltpu.make_async_copy(rowf_v, agg_sh.at[dst_v.at[0]],
                              sems[par]).wait()

    def scale_weights(s):
        for j in range(8):
            idx16 = src_v[s, pl.ds(j * 16, 16)]
            nrm16 = plsc.load_gather(norm_v, [idx16])
            wp_v[pl.ds(j * 16, 16)] = w_v[s, pl.ds(j * 16, 16)] * nrm16

    himask = jnp.full((16,), -65536, jnp.int32)

    def edge_scale(s):
        def edge_body(e, _):
            wsc = plsc.load_gather(wp_v, [jnp.full((16,), e, jnp.int32)])
            for f in range(4):
                bits = rows_v[s, e, pl.ds(f * 16, 16)]
                ev = plsc.bitcast(bits << 16, jnp.float32)
                od = plsc.bitcast(bits & himask, jnp.float32)
                rowf_v[e, pl.ds(f * 32, 16)] = ev * wsc
                rowf_v[e, pl.ds(f * 32 + 16, 16)] = od * wsc
            return 0

        lax.fori_loop(0, 128, edge_body, 0, unroll=8)

    issue_idx(0, 0, base, semi0)
    wait_idx(0, 0, semi0)
    pltpu.async_copy(x_hbm.at[src_v.at[0]], rows_v.at[0], semg0)
    issue_idx(1, 1, base + 1, semi1)

    def quad_body(q, _):
        c0 = base + 4 * q
        for k in range(4):
            s, sn, dk = k % 2, (k + 1) % 2, k
            pltpu.make_async_copy(x_hbm.at[src_v.at[s]], rows_v.at[s],
                                  semg[s]).wait()
            scale_weights(s)
            if k < 2:
                issue_idx(s, (k + 2) % 4, c0 + k + 2, semi[s])
            else:
                @pl.when(q + 1 < n_quads)
                def _():
                    issue_idx(s, (k + 2) % 4, c0 + k + 2, semi[s])
            if k == 3:
                @pl.when(q + 1 < n_quads)
                def _():
                    wait_idx(sn, (k + 1) % 4, semi[sn])
                    pltpu.async_copy(x_hbm.at[src_v.at[sn]], rows_v.at[sn],
                                     semg[sn])
            else:
                wait_idx(sn, (k + 1) % 4, semi[sn])
                pltpu.async_copy(x_hbm.at[src_v.at[sn]], rows_v.at[sn],
                                 semg[sn])
            edge_scale(s)
            pltpu.async_copy(rowf_v, agg_sh.at[dst_v.at[dk]], sems[s],
                             add=True).wait()
        return 0

    lax.fori_loop(0, n_quads, quad_body, 0)
    plsc.subcore_barrier()

    @pl.when(cid == 0)
    def _():
        for b in range(ROWS_PER_TILE // 128):
            off = sid * ROWS_PER_TILE + b * 128
            pltpu.sync_copy(agg_sh.at[pl.ds(off, 128)], out0_hbm.at[pl.ds(off, 128)])

    @pl.when(cid == 1)
    def _():
        for b in range(ROWS_PER_TILE // 128):
            off = sid * ROWS_PER_TILE + b * 128
            pltpu.sync_copy(agg_sh.at[pl.ds(off, 128)], out1_hbm.at[pl.ds(off, 128)])


def _mlp_body(a0_ref, a1_ref, nd_ref, wc_ref, bc_ref, wf_ref, bf_ref,
              w2_ref, b2_ref, out_ref):
    h = (a0_ref[...] + a1_ref[...]) * nd_ref[...]
    h = jnp.dot(h, wc_ref[...], preferred_element_type=jnp.float32) + bc_ref[...]
    h = jnp.maximum(h, 0.0)
    h = jnp.dot(h, wf_ref[...], preferred_element_type=jnp.float32) + bf_ref[...]
    h = jnp.maximum(h, 0.0)
    out_ref[...] = (
        jnp.dot(h, w2_ref[...], preferred_element_type=jnp.float32) + b2_ref[...]
    )


def _mlp(a0, a1, norm_dst, W_conv, b_conv, W_fc, b_fc, W_fc2, b_fc2):
    BR = 1000
    grid = (N_NODES // BR,)
    row_spec = pl.BlockSpec((BR, D), lambda i: (i, 0))
    nd_spec = pl.BlockSpec((BR, 1), lambda i: (i, 0))
    w_spec = pl.BlockSpec((D, D), lambda i: (0, 0))
    b_spec = pl.BlockSpec((1, D), lambda i: (0, 0))
    return pl.pallas_call(
        _mlp_body,
        grid=grid,
        in_specs=[row_spec, row_spec, nd_spec, w_spec, b_spec, w_spec,
                  b_spec, w_spec, b_spec],
        out_specs=row_spec,
        out_shape=jax.ShapeDtypeStruct((N_NODES, D), jnp.float32),
    )(a0, a1, norm_dst, W_conv, b_conv, W_fc, b_fc, W_fc2, b_fc2)


@jax.jit
def kernel(x, edge_index, edge_weight, W_conv, b_conv, W_fc, b_fc, W_fc2,
           b_fc2):
    n_pad_edges = E_ROWS_PAD * 128 - N_EDGES
    src = edge_index[0].astype(jnp.int32)
    dst = edge_index[1].astype(jnp.int32)
    pad_node = jnp.full((n_pad_edges,), N_PAD - 1, jnp.int32)
    src_deg2d = jnp.concatenate([src, pad_node]).reshape(E_ROWS_PAD, 128)
    dst2d = jnp.concatenate([dst, pad_node]).reshape(E_ROWS_PAD, 128)
    src_agg2d = jnp.concatenate(
        [src, jnp.zeros((n_pad_edges,), jnp.int32)]).reshape(E_ROWS_PAD, 128)
    w2d = jnp.concatenate(
        [edge_weight, jnp.zeros((n_pad_edges,), jnp.float32)]
    ).reshape(E_ROWS_PAD, 128)

    x_bf = jax.lax.bitcast_convert_type(
        x[:, _BF16_PERM].astype(jnp.bfloat16).reshape(N_NODES, D // 2, 2),
        jnp.int32)
    deg_partial = _degree_kernel(src_deg2d, dst2d)
    norms = _norms(deg_partial)
    agg0, agg1 = _agg_kernel(x_bf, src_agg2d, dst2d, w2d, norms[0])
    return _mlp(agg0[:N_NODES], agg1[:N_NODES],
                norms[1, :N_NODES, None], W_conv,
                b_conv.reshape(1, D), W_fc, b_fc.reshape(1, D), W_fc2,
                b_fc2.reshape(1, D))
